# baseline (speedup 1.0000x reference)
import functools

import jax
import jax.numpy as jnp
from jax import lax
from jax.experimental import pallas as pl
from jax.experimental.pallas import tpu as pltpu
from jax.experimental.pallas import tpu_sc as plsc

MIN_SCORE = 0.01
SELECT_TOP = 200
NMS_IOU = 0.45

B, N, C = 16, 20000, 80
BLK = 512
NBLK = (B * N) // BLK
K = SELECT_TOP
SLOTS = 208
SLOTSP = 224
NV = N // 16
NPAD = 20480
NW = 20096
OUTW = 256
BIGI = 1 << 28


def _s1_body(s_ref, key_ref, cls_ref):
    s = s_ref[0]
    mx = jnp.max(s, axis=0)
    am = jnp.argmax(s, axis=0).astype(jnp.int32)
    bits = lax.bitcast_convert_type(mx, jnp.uint32)
    sign = bits >> jnp.uint32(31)
    flip = (sign * jnp.uint32(0xFFFFFFFF)) | jnp.uint32(0x80000000)
    keys = lax.bitcast_convert_type(bits ^ flip, jnp.int32)
    key_ref[...] = jnp.concatenate(
        [keys, jnp.zeros((NPAD - N,), jnp.int32)])
    cls_ref[...] = jnp.concatenate([am, jnp.zeros((NPAD - N,), jnp.int32)])


def _stage1(scores_t):
    return pl.pallas_call(
        _s1_body,
        grid=(B,),
        in_specs=[pl.BlockSpec((1, C, N), lambda b: (b, 0, 0))],
        out_specs=[
            pl.BlockSpec((NPAD,), lambda b: (b,)),
            pl.BlockSpec((NPAD,), lambda b: (b,)),
        ],
        out_shape=[
            jax.ShapeDtypeStruct((B * NPAD,), jnp.int32),
            jax.ShapeDtypeStruct((B * NPAD,), jnp.int32),
        ],
    )(scores_t)


def _sc_body(keys_hbm, cls_hbm, boxes_hbm,
             ob_hbm, os_hbm, oc_hbm, ok_hbm,
             keys_v, surv_v, boxes_v, hist_v,
             selk_v, seli_v, skeys_v, sidx_v, sscore_v, valid_v, clssel_v,
             x1_v, y1_v, x2_v, y2_v, area_v, outb_v, keep_v, sup_v):
    cid = lax.axis_index("c")
    sid = lax.axis_index("s")
    wid = sid * 2 + cid

    @pl.when(wid < B)
    def _image():
        b = wid
        lanes = lax.iota(jnp.int32, 16)
        ones16 = jnp.full((16,), 1, jnp.int32)
        zeros16 = jnp.zeros((16,), jnp.int32)

        start = b * NPAD
        pltpu.sync_copy(keys_hbm.at[pl.ds(start, NW)], keys_v)
        pltpu.sync_copy(boxes_hbm.at[pl.ds(b * (N * 4), N * 4)], boxes_v)

        for j in range(13):
            selk_v[pl.ds(j * 16, 16)] = zeros16
            seli_v[pl.ds(j * 16, 16)] = jnp.full((16,), BIGI, jnp.int32)
            skeys_v[pl.ds(j * 16, 16)] = zeros16
            sidx_v[pl.ds(j * 16, 16)] = zeros16

        def _u(ref_slice):
            return plsc.bitcast(ref_slice, jnp.uint32)

        def scan_body(jj, carry):
            cum, k_r, dig_found, done = carry
            j = 15 - jj
            tot = hist_v[pl.ds(j * 16, 16)]
            hist_v[pl.ds(j * 16, 16)] = zeros16
            for l in range(1, 16):
                tot = tot + hist_v[pl.ds(l * 256 + j * 16, 16)]
                hist_v[pl.ds(l * 256 + j * 16, 16)] = zeros16
            rev = lax.rev(tot, (0,))
            csum = plsc.cumsum(rev)
            gcum = csum + cum
            crossed = gcum >= k_r
            anyc = jnp.where(crossed, 1, 0)[15]
            pos = plsc.all_reduce_ffs(crossed)[0]
            hit = (anyc == 1) & (done == 0)
            cb = jnp.max(jnp.where(lanes == pos - 1, gcum, 0))
            cum_before = jnp.where(pos > 0, cb, cum)
            digit = jnp.where(hit, j * 16 + (15 - pos), dig_found)
            k_new = jnp.where(hit, k_r - cum_before, k_r)
            done2 = jnp.where(anyc == 1, 1, done)
            chunk_total = gcum[15]
            cum2 = jnp.where(done == 1, cum,
                             jnp.where(hit, cum, chunk_total))
            return (cum2, k_new, digit, done2)

        def run_scan(k_r0):
            _, k_r, dig, _ = lax.fori_loop(
                0, 16, scan_body,
                (jnp.int32(0), k_r0, jnp.int32(0), jnp.int32(0)))
            return k_r, lax.convert_element_type(dig, jnp.uint32)

        def zero_body(i, _):
            hist_v[pl.ds(i * 16, 16)] = zeros16
            return 0
        lax.fori_loop(0, 256, zero_body, 0)

        def hist0_body(i, _):
            kv = _u(keys_v[pl.ds(i * 16, 16)])
            dig = lax.convert_element_type(kv >> jnp.uint32(24), jnp.int32)
            plsc.addupdate_scatter(hist_v, [lanes * 256 + dig], ones16)
            return 0
        lax.fori_loop(0, NV, hist0_body, 0)
        k_rem, d1 = run_scan(jnp.int32(K))
        prefix = d1 << jnp.uint32(24)

        def hist1_body(i, _, prefix=prefix):
            kv = _u(keys_v[pl.ds(i * 16, 16)])
            match = (kv & jnp.uint32(0xFF000000)) == prefix
            dig = lax.convert_element_type(
                (kv >> jnp.uint32(16)) & jnp.uint32(0xFF), jnp.int32)
            plsc.addupdate_scatter(hist_v, [lanes * 256 + dig],
                                   ones16, mask=match)
            return 0
        lax.fori_loop(0, NV, hist1_body, 0)
        k_rem, d2 = run_scan(k_rem)
        prefix = prefix | (d2 << jnp.uint32(16))
        hi_thr = prefix | jnp.uint32(0x0000FFFF)

        def sweep_body(i, carry):
            sel_base, surv_base = carry
            kv = _u(keys_v[pl.ds(i * 16, 16)])
            ishi = kv > hi_thr
            issv = (kv & jnp.uint32(0xFFFF0000)) == prefix
            nh = plsc.all_reduce_population_count(ishi)[0]
            ns = plsc.all_reduce_population_count(issv)[0]

            @pl.when(nh + ns > 0)
            def _():
                idxv = i * 16 + lanes
                kvi = plsc.bitcast(kv, jnp.int32)
                cs_h = plsc.cumsum(jnp.where(ishi, 1, 0))
                cs_s = plsc.cumsum(jnp.where(issv, 1, 0))
                hpos = sel_base + cs_h - 1
                plsc.store_scatter(seli_v, [hpos], idxv, mask=ishi)
                plsc.store_scatter(selk_v, [hpos], kvi, mask=ishi)
                spos = surv_base + cs_s - 1
                plsc.store_scatter(surv_v, [spos], idxv, mask=issv)
                dig = lax.convert_element_type(
                    (kv >> jnp.uint32(8)) & jnp.uint32(0xFF), jnp.int32)
                plsc.addupdate_scatter(hist_v, [lanes * 256 + dig],
                                       ones16, mask=issv)
            return (sel_base + nh, surv_base + ns)

        c_hi, surv_cnt = lax.fori_loop(0, NV, sweep_body,
                                       (jnp.int32(0), jnp.int32(0)))
        k_rem, d3 = run_scan(k_rem)
        prefix = prefix | (d3 << jnp.uint32(8))

        trip = (surv_cnt + 15) // 16

        def histg_body(i, _, prefix=prefix):
            posm = (i * 16 + lanes) < surv_cnt
            sidx16 = surv_v[pl.ds(i * 16, 16)]
            kv = _u(plsc.load_gather(keys_v, [sidx16], mask=posm))
            match = ((kv & jnp.uint32(0xFFFFFF00)) == prefix) & posm
            dig = lax.convert_element_type(kv & jnp.uint32(0xFF), jnp.int32)
            plsc.addupdate_scatter(hist_v, [lanes * 256 + dig],
                                   ones16, mask=match)
            return 0
        lax.fori_loop(0, trip, histg_body, 0)
        k_rem, d4 = run_scan(k_rem)
        prefix = prefix | d4

        T = prefix
        m0 = jnp.int32(K) - k_rem

        def comp_body(i, carry):
            gt_base, eq_base = carry
            posm = (i * 16 + lanes) < surv_cnt
            sidx16 = surv_v[pl.ds(i * 16, 16)]
            kv = _u(plsc.load_gather(keys_v, [sidx16], mask=posm))
            gt = (kv > T) & posm
            eq = (kv == T) & posm
            ng = plsc.all_reduce_population_count(gt)[0]
            ne = plsc.all_reduce_population_count(eq)[0]

            @pl.when(ng + ne > 0)
            def _():
                kvi = plsc.bitcast(kv, jnp.int32)
                cs_g = plsc.cumsum(jnp.where(gt, 1, 0))
                cs_e = plsc.cumsum(jnp.where(eq, 1, 0))
                gpos = gt_base + cs_g - 1
                epos = m0 + eq_base + cs_e - 1
                plsc.store_scatter(seli_v, [gpos], sidx16, mask=gt)
                plsc.store_scatter(selk_v, [gpos], kvi, mask=gt)
                eqm = eq & (epos < K)
                plsc.store_scatter(seli_v, [epos], sidx16, mask=eqm)
                plsc.store_scatter(selk_v, [epos], kvi, mask=eqm)
            return (gt_base + ng, eq_base + ne)

        lax.fori_loop(0, trip, comp_body, (c_hi, jnp.int32(0)))

        def ext_body(r, _):
            mv = plsc.bitcast(selk_v[pl.ds(0, 16)], jnp.uint32)
            for j in range(1, 13):
                mv = jnp.maximum(
                    mv, plsc.bitcast(selk_v[pl.ds(j * 16, 16)], jnp.uint32))
            M = jnp.max(mv)
            iv = jnp.full((16,), BIGI, jnp.int32)
            for j in range(13):
                kj = plsc.bitcast(selk_v[pl.ds(j * 16, 16)], jnp.uint32)
                ij = seli_v[pl.ds(j * 16, 16)]
                iv = jnp.minimum(iv, jnp.where(kj == M, ij, BIGI))
            I = jnp.min(iv)
            for j in range(13):
                kj = plsc.bitcast(selk_v[pl.ds(j * 16, 16)], jnp.uint32)
                ij = seli_v[pl.ds(j * 16, 16)]
                kill = (kj == M) & (ij == I)
                selk_v[pl.ds(j * 16, 16)] = plsc.bitcast(
                    jnp.where(kill, jnp.uint32(0), kj), jnp.int32)
                seli_v[pl.ds(j * 16, 16)] = jnp.where(kill, BIGI, ij)
            lane0 = lanes == 0
            rvec = jnp.full((16,), r, jnp.int32)
            mvec = plsc.bitcast(jnp.broadcast_to(M, (16,)), jnp.int32)
            ivec = jnp.broadcast_to(jnp.where(I == BIGI, 0, I), (16,))
            plsc.store_scatter(skeys_v, [rvec], mvec, mask=lane0)
            plsc.store_scatter(sidx_v, [rvec], ivec, mask=lane0)
            return 0

        lax.fori_loop(0, K, ext_body, 0)

        for j in range(13):
            u = plsc.bitcast(skeys_v[pl.ds(j * 16, 16)], jnp.uint32)
            top = u >> jnp.uint32(31)
            flip = jnp.where(top == jnp.uint32(1),
                             jnp.uint32(0x80000000), jnp.uint32(0xFFFFFFFF))
            f = plsc.bitcast(u ^ flip, jnp.float32)
            sscore_v[pl.ds(j * 16, 16)] = f
            valid_v[pl.ds(j * 16, 16)] = jnp.where(f > MIN_SCORE, 1, 0)

        pltpu.sync_copy(cls_hbm.at[pl.ds(start, NW)], surv_v)
        mc = jnp.full((16,), -3.0e38, jnp.float32)
        for j in range(13):
            rows = sidx_v[pl.ds(j * 16, 16)]
            cg = plsc.load_gather(surv_v, [rows]) + 1
            clssel_v[pl.ds(j * 16, 16)] = cg
            slot = j * 16 + lanes
            smask = slot < K
            x1 = plsc.load_gather(boxes_v, [rows])
            y1 = plsc.load_gather(boxes_v, [rows + N])
            x2 = plsc.load_gather(boxes_v, [rows + 2 * N])
            y2 = plsc.load_gather(boxes_v, [rows + 3 * N])
            x1_v[pl.ds(j * 16, 16)] = x1
            y1_v[pl.ds(j * 16, 16)] = y1
            x2_v[pl.ds(j * 16, 16)] = x2
            y2_v[pl.ds(j * 16, 16)] = y2
            s4 = slot * 4
            plsc.store_scatter(outb_v, [s4], x1, mask=smask)
            plsc.store_scatter(outb_v, [s4 + 1], y1, mask=smask)
            plsc.store_scatter(outb_v, [s4 + 2], x2, mask=smask)
            plsc.store_scatter(outb_v, [s4 + 3], y2, mask=smask)
            neg = jnp.full((16,), -3.0e38, jnp.float32)
            mc = jnp.maximum(mc, jnp.where(smask, x1, neg))
            mc = jnp.maximum(mc, jnp.where(smask, y1, neg))
            mc = jnp.maximum(mc, jnp.where(smask, x2, neg))
            mc = jnp.maximum(mc, jnp.where(smask, y2, neg))
        off_scale = jnp.max(mc) + jnp.float32(1.0)

        for j in range(13):
            offs = clssel_v[pl.ds(j * 16, 16)].astype(jnp.float32) * off_scale
            x1o = x1_v[pl.ds(j * 16, 16)] + offs
            y1o = y1_v[pl.ds(j * 16, 16)] + offs
            x2o = x2_v[pl.ds(j * 16, 16)] + offs
            y2o = y2_v[pl.ds(j * 16, 16)] + offs
            x1_v[pl.ds(j * 16, 16)] = x1o
            y1_v[pl.ds(j * 16, 16)] = y1o
            x2_v[pl.ds(j * 16, 16)] = x2o
            y2_v[pl.ds(j * 16, 16)] = y2o
            area_v[pl.ds(j * 16, 16)] = (
                jnp.maximum(x2o - x1o, 0.0) * jnp.maximum(y2o - y1o, 0.0))
            sup_v[pl.ds(j * 16, 16)] = zeros16
            keep_v[pl.ds(j * 16, 16)] = zeros16

        def nms_body(r, _):
            kflag = ((sup_v[pl.ds(r, 16)][0] == 0) &
                     (valid_v[pl.ds(r, 16)][0] == 1))
            kvec = jnp.broadcast_to(jnp.where(kflag, 1, 0), (16,))
            plsc.store_scatter(keep_v, [jnp.full((16,), r, jnp.int32)],
                               kvec, mask=lanes == 0)

            @pl.when(kflag)
            def _():
                x1i = x1_v[pl.ds(r, 16)][0]
                y1i = y1_v[pl.ds(r, 16)][0]
                x2i = x2_v[pl.ds(r, 16)][0]
                y2i = y2_v[pl.ds(r, 16)][0]
                ai = area_v[pl.ds(r, 16)][0]

                def row_body(cj, _):
                    c0 = cj * 16
                    xx1 = jnp.maximum(x1_v[pl.ds(c0, 16)], x1i)
                    yy1 = jnp.maximum(y1_v[pl.ds(c0, 16)], y1i)
                    xx2 = jnp.minimum(x2_v[pl.ds(c0, 16)], x2i)
                    yy2 = jnp.minimum(y2_v[pl.ds(c0, 16)], y2i)
                    inter = (jnp.maximum(xx2 - xx1, 0.0) *
                             jnp.maximum(yy2 - yy1, 0.0))
                    union = ai + area_v[pl.ds(c0, 16)] - inter
                    iou = inter / jnp.maximum(union, 1e-9)
                    supn = jnp.where(iou > NMS_IOU, 1, 0)
                    sup_v[pl.ds(c0, 16)] = sup_v[pl.ds(c0, 16)] | supn
                    return 0

                lax.fori_loop(r // 16, 13, row_body, 0)
            return 0

        lax.fori_loop(0, K, nms_body, 0)

        pltpu.sync_copy(outb_v, ob_hbm.at[pl.ds(b * 1024, 1024)])
        pltpu.sync_copy(sscore_v, os_hbm.at[pl.ds(b * OUTW, OUTW)])
        pltpu.sync_copy(clssel_v, oc_hbm.at[pl.ds(b * OUTW, OUTW)])
        pltpu.sync_copy(keep_v, ok_hbm.at[pl.ds(b * OUTW, OUTW)])


def _stage2(keys, cls, boxes):
    mesh = plsc.VectorSubcoreMesh(core_axis_name="c", subcore_axis_name="s",
                                  num_cores=2, num_subcores=16)
    f = functools.partial(
        pl.kernel,
        out_type=[
            jax.ShapeDtypeStruct((B * 1024,), jnp.float32),
            jax.ShapeDtypeStruct((B * OUTW,), jnp.float32),
            jax.ShapeDtypeStruct((B * OUTW,), jnp.int32),
            jax.ShapeDtypeStruct((B * OUTW,), jnp.int32),
        ],
        mesh=mesh,
        compiler_params=pltpu.CompilerParams(needs_layout_passes=False),
        scratch_types=[
            pltpu.VMEM((NW,), jnp.int32),
            pltpu.VMEM((NW,), jnp.int32),
            pltpu.VMEM((N * 4,), jnp.float32),
            pltpu.VMEM((4096,), jnp.int32),
            pltpu.VMEM((SLOTSP,), jnp.int32),
            pltpu.VMEM((SLOTSP,), jnp.int32),
            pltpu.VMEM((SLOTSP,), jnp.int32),
            pltpu.VMEM((SLOTSP,), jnp.int32),
            pltpu.VMEM((OUTW,), jnp.float32),
            pltpu.VMEM((SLOTSP,), jnp.int32),
            pltpu.VMEM((OUTW,), jnp.int32),
            pltpu.VMEM((SLOTSP,), jnp.float32),
            pltpu.VMEM((SLOTSP,), jnp.float32),
            pltpu.VMEM((SLOTSP,), jnp.float32),
            pltpu.VMEM((SLOTSP,), jnp.float32),
            pltpu.VMEM((SLOTSP,), jnp.float32),
            pltpu.VMEM((1024,), jnp.float32),
            pltpu.VMEM((OUTW,), jnp.int32),
            pltpu.VMEM((SLOTSP,), jnp.int32),
        ],
    )(_sc_body)
    return f(keys, cls, boxes)


def kernel(batched_bboxes, batched_scores):
    scores_t = batched_scores.transpose(0, 2, 1)
    keys, cls = _stage1(scores_t)
    boxes_flat = batched_bboxes.transpose(0, 2, 1).reshape(B * N * 4)
    ob, osc, ocl, okp = _stage2(keys, cls, boxes_flat)
    ob = ob.reshape(B, OUTW, 4)[:, :K, :]
    osc = osc.reshape(B, OUTW)[:, :K]
    ocl = ocl.reshape(B, OUTW)[:, :K]
    okp = okp.reshape(B, OUTW)[:, :K]
    return ob, osc, ocl, okp.astype(bool)


# speedup vs baseline: 1.1041x; 1.1041x over previous
import functools

import jax
import jax.numpy as jnp
from jax import lax
from jax.experimental import pallas as pl
from jax.experimental.pallas import tpu as pltpu
from jax.experimental.pallas import tpu_sc as plsc

MIN_SCORE = 0.01
SELECT_TOP = 200
NMS_IOU = 0.45

B, N, C = 16, 20000, 80
BLK = 512
NBLK = (B * N) // BLK
K = SELECT_TOP
SLOTS = 208
SLOTSP = 224
NV = N // 16
NPAD = 20480
NW = 20096
OUTW = 256
BIGI = 1 << 28


def _s1_body(s_ref, key_ref, cls_ref):
    s = s_ref[0]
    mx = jnp.max(s, axis=0)
    am = jnp.argmax(s, axis=0).astype(jnp.int32)
    bits = lax.bitcast_convert_type(mx, jnp.uint32)
    sign = bits >> jnp.uint32(31)
    flip = (sign * jnp.uint32(0xFFFFFFFF)) | jnp.uint32(0x80000000)
    keys = lax.bitcast_convert_type(bits ^ flip, jnp.int32)
    key_ref[...] = jnp.concatenate(
        [keys, jnp.zeros((NPAD - N,), jnp.int32)])
    cls_ref[...] = jnp.concatenate([am, jnp.zeros((NPAD - N,), jnp.int32)])


def _stage1(scores_t):
    return pl.pallas_call(
        _s1_body,
        grid=(B,),
        in_specs=[pl.BlockSpec((1, C, N), lambda b: (b, 0, 0))],
        out_specs=[
            pl.BlockSpec((NPAD,), lambda b: (b,)),
            pl.BlockSpec((NPAD,), lambda b: (b,)),
        ],
        out_shape=[
            jax.ShapeDtypeStruct((B * NPAD,), jnp.int32),
            jax.ShapeDtypeStruct((B * NPAD,), jnp.int32),
        ],
    )(scores_t)


def _sc_body(keys_hbm, cls_hbm, boxes_hbm,
             ob_hbm, os_hbm, oc_hbm, ok_hbm,
             keys_v, surv_v, boxes_v, hist_v,
             selk_v, seli_v, skeys_v, sidx_v, sscore_v, valid_v, clssel_v,
             x1_v, y1_v, x2_v, y2_v, area_v, outb_v, keep_v, sup_v):
    cid = lax.axis_index("c")
    sid = lax.axis_index("s")
    wid = sid * 2 + cid

    @pl.when(wid < B)
    def _image():
        b = wid
        lanes = lax.iota(jnp.int32, 16)
        ones16 = jnp.full((16,), 1, jnp.int32)
        zeros16 = jnp.zeros((16,), jnp.int32)

        start = b * NPAD
        pltpu.sync_copy(keys_hbm.at[pl.ds(start, NW)], keys_v)
        pltpu.sync_copy(boxes_hbm.at[pl.ds(b * (N * 4), N * 4)], boxes_v)

        for j in range(13):
            selk_v[pl.ds(j * 16, 16)] = zeros16
            seli_v[pl.ds(j * 16, 16)] = jnp.full((16,), BIGI, jnp.int32)
            skeys_v[pl.ds(j * 16, 16)] = zeros16
            sidx_v[pl.ds(j * 16, 16)] = zeros16

        def _u(ref_slice):
            return plsc.bitcast(ref_slice, jnp.uint32)

        def scan_body(jj, carry):
            cum, k_r, dig_found, done = carry
            j = 15 - jj
            tot = hist_v[pl.ds(j * 16, 16)]
            hist_v[pl.ds(j * 16, 16)] = zeros16
            for l in range(1, 16):
                tot = tot + hist_v[pl.ds(l * 256 + j * 16, 16)]
                hist_v[pl.ds(l * 256 + j * 16, 16)] = zeros16
            rev = lax.rev(tot, (0,))
            csum = plsc.cumsum(rev)
            gcum = csum + cum
            crossed = gcum >= k_r
            anyc = jnp.where(crossed, 1, 0)[15]
            pos = plsc.all_reduce_ffs(crossed)[0]
            hit = (anyc == 1) & (done == 0)
            cb = jnp.max(jnp.where(lanes == pos - 1, gcum, 0))
            cum_before = jnp.where(pos > 0, cb, cum)
            digit = jnp.where(hit, j * 16 + (15 - pos), dig_found)
            k_new = jnp.where(hit, k_r - cum_before, k_r)
            done2 = jnp.where(anyc == 1, 1, done)
            chunk_total = gcum[15]
            cum2 = jnp.where(done == 1, cum,
                             jnp.where(hit, cum, chunk_total))
            return (cum2, k_new, digit, done2)

        def run_scan(k_r0):
            _, k_r, dig, _ = lax.fori_loop(
                0, 16, scan_body,
                (jnp.int32(0), k_r0, jnp.int32(0), jnp.int32(0)))
            return k_r, lax.convert_element_type(dig, jnp.uint32)

        def zero_body(i, _):
            hist_v[pl.ds(i * 16, 16)] = zeros16
            return 0
        lax.fori_loop(0, 256, zero_body, 0)

        def hist0_body(i, _):
            kv = _u(keys_v[pl.ds(i * 16, 16)])
            dig = lax.convert_element_type(kv >> jnp.uint32(24), jnp.int32)
            plsc.addupdate_scatter(hist_v, [lanes * 256 + dig], ones16)
            return 0
        lax.fori_loop(0, NV, hist0_body, 0, unroll=4)
        k_rem, d1 = run_scan(jnp.int32(K))
        prefix = d1 << jnp.uint32(24)

        def hist1_body(i, _, prefix=prefix):
            kv = _u(keys_v[pl.ds(i * 16, 16)])
            match = (kv & jnp.uint32(0xFF000000)) == prefix
            dig = lax.convert_element_type(
                (kv >> jnp.uint32(16)) & jnp.uint32(0xFF), jnp.int32)
            plsc.addupdate_scatter(hist_v, [lanes * 256 + dig],
                                   ones16, mask=match)
            return 0
        lax.fori_loop(0, NV, hist1_body, 0, unroll=4)
        k_rem, d2 = run_scan(k_rem)
        prefix = prefix | (d2 << jnp.uint32(16))
        hi_thr = prefix | jnp.uint32(0x0000FFFF)

        def sweep_body(i, carry):
            sel_base, surv_base = carry
            kv = _u(keys_v[pl.ds(i * 16, 16)])
            ishi = kv > hi_thr
            issv = (kv & jnp.uint32(0xFFFF0000)) == prefix
            nh = plsc.all_reduce_population_count(ishi)[0]
            ns = plsc.all_reduce_population_count(issv)[0]

            @pl.when(nh + ns > 0)
            def _():
                idxv = i * 16 + lanes
                kvi = plsc.bitcast(kv, jnp.int32)
                cs_h = plsc.cumsum(jnp.where(ishi, 1, 0))
                cs_s = plsc.cumsum(jnp.where(issv, 1, 0))
                hpos = sel_base + cs_h - 1
                plsc.store_scatter(seli_v, [hpos], idxv, mask=ishi)
                plsc.store_scatter(selk_v, [hpos], kvi, mask=ishi)
                spos = surv_base + cs_s - 1
                plsc.store_scatter(surv_v, [spos], idxv, mask=issv)
                dig = lax.convert_element_type(
                    (kv >> jnp.uint32(8)) & jnp.uint32(0xFF), jnp.int32)
                plsc.addupdate_scatter(hist_v, [lanes * 256 + dig],
                                       ones16, mask=issv)
            return (sel_base + nh, surv_base + ns)

        c_hi, surv_cnt = lax.fori_loop(0, NV, sweep_body,
                                       (jnp.int32(0), jnp.int32(0)))
        k_rem, d3 = run_scan(k_rem)
        prefix = prefix | (d3 << jnp.uint32(8))

        trip = (surv_cnt + 15) // 16

        def histg_body(i, _, prefix=prefix):
            posm = (i * 16 + lanes) < surv_cnt
            sidx16 = surv_v[pl.ds(i * 16, 16)]
            kv = _u(plsc.load_gather(keys_v, [sidx16], mask=posm))
            match = ((kv & jnp.uint32(0xFFFFFF00)) == prefix) & posm
            dig = lax.convert_element_type(kv & jnp.uint32(0xFF), jnp.int32)
            plsc.addupdate_scatter(hist_v, [lanes * 256 + dig],
                                   ones16, mask=match)
            return 0
        lax.fori_loop(0, trip, histg_body, 0)
        k_rem, d4 = run_scan(k_rem)
        prefix = prefix | d4

        T = prefix
        m0 = jnp.int32(K) - k_rem

        def comp_body(i, carry):
            gt_base, eq_base = carry
            posm = (i * 16 + lanes) < surv_cnt
            sidx16 = surv_v[pl.ds(i * 16, 16)]
            kv = _u(plsc.load_gather(keys_v, [sidx16], mask=posm))
            gt = (kv > T) & posm
            eq = (kv == T) & posm
            ng = plsc.all_reduce_population_count(gt)[0]
            ne = plsc.all_reduce_population_count(eq)[0]

            @pl.when(ng + ne > 0)
            def _():
                kvi = plsc.bitcast(kv, jnp.int32)
                cs_g = plsc.cumsum(jnp.where(gt, 1, 0))
                cs_e = plsc.cumsum(jnp.where(eq, 1, 0))
                gpos = gt_base + cs_g - 1
                epos = m0 + eq_base + cs_e - 1
                plsc.store_scatter(seli_v, [gpos], sidx16, mask=gt)
                plsc.store_scatter(selk_v, [gpos], kvi, mask=gt)
                eqm = eq & (epos < K)
                plsc.store_scatter(seli_v, [epos], sidx16, mask=eqm)
                plsc.store_scatter(selk_v, [epos], kvi, mask=eqm)
            return (gt_base + ng, eq_base + ne)

        lax.fori_loop(0, trip, comp_body, (c_hi, jnp.int32(0)))

        def ext_body(r, _):
            mv = plsc.bitcast(selk_v[pl.ds(0, 16)], jnp.uint32)
            for j in range(1, 13):
                mv = jnp.maximum(
                    mv, plsc.bitcast(selk_v[pl.ds(j * 16, 16)], jnp.uint32))
            M = jnp.max(mv)
            iv = jnp.full((16,), BIGI, jnp.int32)
            for j in range(13):
                kj = plsc.bitcast(selk_v[pl.ds(j * 16, 16)], jnp.uint32)
                ij = seli_v[pl.ds(j * 16, 16)]
                iv = jnp.minimum(iv, jnp.where(kj == M, ij, BIGI))
            I = jnp.min(iv)
            for j in range(13):
                kj = plsc.bitcast(selk_v[pl.ds(j * 16, 16)], jnp.uint32)
                ij = seli_v[pl.ds(j * 16, 16)]
                kill = (kj == M) & (ij == I)
                selk_v[pl.ds(j * 16, 16)] = plsc.bitcast(
                    jnp.where(kill, jnp.uint32(0), kj), jnp.int32)
                seli_v[pl.ds(j * 16, 16)] = jnp.where(kill, BIGI, ij)
            lane0 = lanes == 0
            rvec = jnp.full((16,), r, jnp.int32)
            mvec = plsc.bitcast(jnp.broadcast_to(M, (16,)), jnp.int32)
            ivec = jnp.broadcast_to(jnp.where(I == BIGI, 0, I), (16,))
            plsc.store_scatter(skeys_v, [rvec], mvec, mask=lane0)
            plsc.store_scatter(sidx_v, [rvec], ivec, mask=lane0)
            return 0

        lax.fori_loop(0, K, ext_body, 0, unroll=2)

        for j in range(13):
            u = plsc.bitcast(skeys_v[pl.ds(j * 16, 16)], jnp.uint32)
            top = u >> jnp.uint32(31)
            flip = jnp.where(top == jnp.uint32(1),
                             jnp.uint32(0x80000000), jnp.uint32(0xFFFFFFFF))
            f = plsc.bitcast(u ^ flip, jnp.float32)
            sscore_v[pl.ds(j * 16, 16)] = f
            valid_v[pl.ds(j * 16, 16)] = jnp.where(f > MIN_SCORE, 1, 0)

        pltpu.sync_copy(cls_hbm.at[pl.ds(start, NW)], surv_v)
        mc = jnp.full((16,), -3.0e38, jnp.float32)
        for j in range(13):
            rows = sidx_v[pl.ds(j * 16, 16)]
            cg = plsc.load_gather(surv_v, [rows]) + 1
            clssel_v[pl.ds(j * 16, 16)] = cg
            slot = j * 16 + lanes
            smask = slot < K
            x1 = plsc.load_gather(boxes_v, [rows])
            y1 = plsc.load_gather(boxes_v, [rows + N])
            x2 = plsc.load_gather(boxes_v, [rows + 2 * N])
            y2 = plsc.load_gather(boxes_v, [rows + 3 * N])
            x1_v[pl.ds(j * 16, 16)] = x1
            y1_v[pl.ds(j * 16, 16)] = y1
            x2_v[pl.ds(j * 16, 16)] = x2
            y2_v[pl.ds(j * 16, 16)] = y2
            s4 = slot * 4
            plsc.store_scatter(outb_v, [s4], x1, mask=smask)
            plsc.store_scatter(outb_v, [s4 + 1], y1, mask=smask)
            plsc.store_scatter(outb_v, [s4 + 2], x2, mask=smask)
            plsc.store_scatter(outb_v, [s4 + 3], y2, mask=smask)
            neg = jnp.full((16,), -3.0e38, jnp.float32)
            mc = jnp.maximum(mc, jnp.where(smask, x1, neg))
            mc = jnp.maximum(mc, jnp.where(smask, y1, neg))
            mc = jnp.maximum(mc, jnp.where(smask, x2, neg))
            mc = jnp.maximum(mc, jnp.where(smask, y2, neg))
        off_scale = jnp.max(mc) + jnp.float32(1.0)

        for j in range(13):
            offs = clssel_v[pl.ds(j * 16, 16)].astype(jnp.float32) * off_scale
            x1o = x1_v[pl.ds(j * 16, 16)] + offs
            y1o = y1_v[pl.ds(j * 16, 16)] + offs
            x2o = x2_v[pl.ds(j * 16, 16)] + offs
            y2o = y2_v[pl.ds(j * 16, 16)] + offs
            x1_v[pl.ds(j * 16, 16)] = x1o
            y1_v[pl.ds(j * 16, 16)] = y1o
            x2_v[pl.ds(j * 16, 16)] = x2o
            y2_v[pl.ds(j * 16, 16)] = y2o
            area_v[pl.ds(j * 16, 16)] = (
                jnp.maximum(x2o - x1o, 0.0) * jnp.maximum(y2o - y1o, 0.0))
            sup_v[pl.ds(j * 16, 16)] = zeros16
            keep_v[pl.ds(j * 16, 16)] = zeros16

        def nms_body(r, _):
            kflag = ((sup_v[pl.ds(r, 16)][0] == 0) &
                     (valid_v[pl.ds(r, 16)][0] == 1))
            kvec = jnp.broadcast_to(jnp.where(kflag, 1, 0), (16,))
            plsc.store_scatter(keep_v, [jnp.full((16,), r, jnp.int32)],
                               kvec, mask=lanes == 0)

            @pl.when(kflag)
            def _():
                x1i = x1_v[pl.ds(r, 16)][0]
                y1i = y1_v[pl.ds(r, 16)][0]
                x2i = x2_v[pl.ds(r, 16)][0]
                y2i = y2_v[pl.ds(r, 16)][0]
                ai = area_v[pl.ds(r, 16)][0]

                for j in range(13):
                    c0 = j * 16
                    xx1 = jnp.maximum(x1_v[pl.ds(c0, 16)], x1i)
                    yy1 = jnp.maximum(y1_v[pl.ds(c0, 16)], y1i)
                    xx2 = jnp.minimum(x2_v[pl.ds(c0, 16)], x2i)
                    yy2 = jnp.minimum(y2_v[pl.ds(c0, 16)], y2i)
                    inter = (jnp.maximum(xx2 - xx1, 0.0) *
                             jnp.maximum(yy2 - yy1, 0.0))
                    union = ai + area_v[pl.ds(c0, 16)] - inter
                    iou = inter / jnp.maximum(union, 1e-9)
                    supn = jnp.where(iou > NMS_IOU, 1, 0)
                    sup_v[pl.ds(c0, 16)] = sup_v[pl.ds(c0, 16)] | supn
            return 0

        lax.fori_loop(0, K, nms_body, 0, unroll=2)

        pltpu.sync_copy(outb_v, ob_hbm.at[pl.ds(b * 1024, 1024)])
        pltpu.sync_copy(sscore_v, os_hbm.at[pl.ds(b * OUTW, OUTW)])
        pltpu.sync_copy(clssel_v, oc_hbm.at[pl.ds(b * OUTW, OUTW)])
        pltpu.sync_copy(keep_v, ok_hbm.at[pl.ds(b * OUTW, OUTW)])


def _stage2(keys, cls, boxes):
    mesh = plsc.VectorSubcoreMesh(core_axis_name="c", subcore_axis_name="s",
                                  num_cores=2, num_subcores=16)
    f = functools.partial(
        pl.kernel,
        out_type=[
            jax.ShapeDtypeStruct((B * 1024,), jnp.float32),
            jax.ShapeDtypeStruct((B * OUTW,), jnp.float32),
            jax.ShapeDtypeStruct((B * OUTW,), jnp.int32),
            jax.ShapeDtypeStruct((B * OUTW,), jnp.int32),
        ],
        mesh=mesh,
        compiler_params=pltpu.CompilerParams(needs_layout_passes=False),
        scratch_types=[
            pltpu.VMEM((NW,), jnp.int32),
            pltpu.VMEM((NW,), jnp.int32),
            pltpu.VMEM((N * 4,), jnp.float32),
            pltpu.VMEM((4096,), jnp.int32),
            pltpu.VMEM((SLOTSP,), jnp.int32),
            pltpu.VMEM((SLOTSP,), jnp.int32),
            pltpu.VMEM((SLOTSP,), jnp.int32),
            pltpu.VMEM((SLOTSP,), jnp.int32),
            pltpu.VMEM((OUTW,), jnp.float32),
            pltpu.VMEM((SLOTSP,), jnp.int32),
            pltpu.VMEM((OUTW,), jnp.int32),
            pltpu.VMEM((SLOTSP,), jnp.float32),
            pltpu.VMEM((SLOTSP,), jnp.float32),
            pltpu.VMEM((SLOTSP,), jnp.float32),
            pltpu.VMEM((SLOTSP,), jnp.float32),
            pltpu.VMEM((SLOTSP,), jnp.float32),
            pltpu.VMEM((1024,), jnp.float32),
            pltpu.VMEM((OUTW,), jnp.int32),
            pltpu.VMEM((SLOTSP,), jnp.int32),
        ],
    )(_sc_body)
    return f(keys, cls, boxes)


def kernel(batched_bboxes, batched_scores):
    scores_t = batched_scores.transpose(0, 2, 1)
    keys, cls = _stage1(scores_t)
    boxes_flat = batched_bboxes.transpose(0, 2, 1).reshape(B * N * 4)
    ob, osc, ocl, okp = _stage2(keys, cls, boxes_flat)
    ob = ob.reshape(B, OUTW, 4)[:, :K, :]
    osc = osc.reshape(B, OUTW)[:, :K]
    ocl = ocl.reshape(B, OUTW)[:, :K]
    okp = okp.reshape(B, OUTW)[:, :K]
    return ob, osc, ocl, okp.astype(bool)


# speedup vs baseline: 1.1055x; 1.0012x over previous
import functools

import jax
import jax.numpy as jnp
from jax import lax
from jax.experimental import pallas as pl
from jax.experimental.pallas import tpu as pltpu
from jax.experimental.pallas import tpu_sc as plsc

MIN_SCORE = 0.01
SELECT_TOP = 200
NMS_IOU = 0.45

B, N, C = 16, 20000, 80
BLK = 512
NBLK = (B * N) // BLK
K = SELECT_TOP
SLOTS = 208
SLOTSP = 224
NV = N // 16
NPAD = 20480
NW = 20096
OUTW = 256
BIGI = 1 << 28


def _s1_body(s_ref, key_ref, cls_ref):
    s = s_ref[0]
    mx = jnp.max(s, axis=0)
    am = jnp.argmax(s, axis=0).astype(jnp.int32)
    bits = lax.bitcast_convert_type(mx, jnp.uint32)
    sign = bits >> jnp.uint32(31)
    flip = (sign * jnp.uint32(0xFFFFFFFF)) | jnp.uint32(0x80000000)
    keys = lax.bitcast_convert_type(bits ^ flip, jnp.int32)
    key_ref[...] = jnp.concatenate(
        [keys, jnp.zeros((NPAD - N,), jnp.int32)])
    cls_ref[...] = jnp.concatenate([am, jnp.zeros((NPAD - N,), jnp.int32)])


def _stage1(scores_t):
    return pl.pallas_call(
        _s1_body,
        grid=(B,),
        in_specs=[pl.BlockSpec((1, C, N), lambda b: (b, 0, 0))],
        out_specs=[
            pl.BlockSpec((NPAD,), lambda b: (b,)),
            pl.BlockSpec((NPAD,), lambda b: (b,)),
        ],
        out_shape=[
            jax.ShapeDtypeStruct((B * NPAD,), jnp.int32),
            jax.ShapeDtypeStruct((B * NPAD,), jnp.int32),
        ],
    )(scores_t)


def _sc_body(keys_hbm, cls_hbm, boxes_hbm,
             ob_hbm, os_hbm, oc_hbm, ok_hbm,
             keys_v, surv_v, boxes_v, hist_v,
             selk_v, seli_v, skeys_v, sidx_v, sscore_v, valid_v, clssel_v,
             x1_v, y1_v, x2_v, y2_v, area_v, outb_v, keep_v, sup_v):
    cid = lax.axis_index("c")
    sid = lax.axis_index("s")
    wid = sid * 2 + cid

    @pl.when(wid < B)
    def _image():
        b = wid
        lanes = lax.iota(jnp.int32, 16)
        ones16 = jnp.full((16,), 1, jnp.int32)
        zeros16 = jnp.zeros((16,), jnp.int32)

        start = b * NPAD
        pltpu.sync_copy(keys_hbm.at[pl.ds(start, NW)], keys_v)
        pltpu.sync_copy(boxes_hbm.at[pl.ds(b * (N * 4), N * 4)], boxes_v)

        for j in range(13):
            selk_v[pl.ds(j * 16, 16)] = zeros16
            seli_v[pl.ds(j * 16, 16)] = jnp.full((16,), BIGI, jnp.int32)
            skeys_v[pl.ds(j * 16, 16)] = zeros16
            sidx_v[pl.ds(j * 16, 16)] = zeros16

        def _u(ref_slice):
            return plsc.bitcast(ref_slice, jnp.uint32)

        def scan_body(jj, carry):
            cum, k_r, dig_found, done = carry
            j = 15 - jj
            tot = hist_v[pl.ds(j * 16, 16)]
            hist_v[pl.ds(j * 16, 16)] = zeros16
            for l in range(1, 16):
                tot = tot + hist_v[pl.ds(l * 256 + j * 16, 16)]
                hist_v[pl.ds(l * 256 + j * 16, 16)] = zeros16
            rev = lax.rev(tot, (0,))
            csum = plsc.cumsum(rev)
            gcum = csum + cum
            crossed = gcum >= k_r
            anyc = jnp.where(crossed, 1, 0)[15]
            pos = plsc.all_reduce_ffs(crossed)[0]
            hit = (anyc == 1) & (done == 0)
            cb = jnp.max(jnp.where(lanes == pos - 1, gcum, 0))
            cum_before = jnp.where(pos > 0, cb, cum)
            digit = jnp.where(hit, j * 16 + (15 - pos), dig_found)
            k_new = jnp.where(hit, k_r - cum_before, k_r)
            done2 = jnp.where(anyc == 1, 1, done)
            chunk_total = gcum[15]
            cum2 = jnp.where(done == 1, cum,
                             jnp.where(hit, cum, chunk_total))
            return (cum2, k_new, digit, done2)

        def run_scan(k_r0):
            _, k_r, dig, _ = lax.fori_loop(
                0, 16, scan_body,
                (jnp.int32(0), k_r0, jnp.int32(0), jnp.int32(0)))
            return k_r, lax.convert_element_type(dig, jnp.uint32)

        def zero_body(i, _):
            hist_v[pl.ds(i * 16, 16)] = zeros16
            return 0
        lax.fori_loop(0, 256, zero_body, 0, unroll=8)

        def hist0_body(i, _):
            kv = _u(keys_v[pl.ds(i * 16, 16)])
            dig = lax.convert_element_type(kv >> jnp.uint32(24), jnp.int32)
            plsc.addupdate_scatter(hist_v, [lanes * 256 + dig], ones16)
            return 0
        lax.fori_loop(0, NV, hist0_body, 0, unroll=8)
        k_rem, d1 = run_scan(jnp.int32(K))
        prefix = d1 << jnp.uint32(24)

        def hist1_body(i, _, prefix=prefix):
            kv = _u(keys_v[pl.ds(i * 16, 16)])
            match = (kv & jnp.uint32(0xFF000000)) == prefix
            dig = lax.convert_element_type(
                (kv >> jnp.uint32(16)) & jnp.uint32(0xFF), jnp.int32)
            plsc.addupdate_scatter(hist_v, [lanes * 256 + dig],
                                   ones16, mask=match)
            return 0
        lax.fori_loop(0, NV, hist1_body, 0, unroll=8)
        k_rem, d2 = run_scan(k_rem)
        prefix = prefix | (d2 << jnp.uint32(16))
        hi_thr = prefix | jnp.uint32(0x0000FFFF)

        def sweep_body(i, carry):
            sel_base, surv_base = carry
            kv = _u(keys_v[pl.ds(i * 16, 16)])
            ishi = kv > hi_thr
            issv = (kv & jnp.uint32(0xFFFF0000)) == prefix
            nh = plsc.all_reduce_population_count(ishi)[0]
            ns = plsc.all_reduce_population_count(issv)[0]

            @pl.when(nh + ns > 0)
            def _():
                idxv = i * 16 + lanes
                kvi = plsc.bitcast(kv, jnp.int32)
                cs_h = plsc.cumsum(jnp.where(ishi, 1, 0))
                cs_s = plsc.cumsum(jnp.where(issv, 1, 0))
                hpos = sel_base + cs_h - 1
                plsc.store_scatter(seli_v, [hpos], idxv, mask=ishi)
                plsc.store_scatter(selk_v, [hpos], kvi, mask=ishi)
                spos = surv_base + cs_s - 1
                plsc.store_scatter(surv_v, [spos], idxv, mask=issv)
                dig = lax.convert_element_type(
                    (kv >> jnp.uint32(8)) & jnp.uint32(0xFF), jnp.int32)
                plsc.addupdate_scatter(hist_v, [lanes * 256 + dig],
                                       ones16, mask=issv)
            return (sel_base + nh, surv_base + ns)

        c_hi, surv_cnt = lax.fori_loop(0, NV, sweep_body,
                                       (jnp.int32(0), jnp.int32(0)),
                                       unroll=4)
        k_rem, d3 = run_scan(k_rem)
        prefix = prefix | (d3 << jnp.uint32(8))

        trip = (surv_cnt + 15) // 16

        def histg_body(i, _, prefix=prefix):
            posm = (i * 16 + lanes) < surv_cnt
            sidx16 = surv_v[pl.ds(i * 16, 16)]
            kv = _u(plsc.load_gather(keys_v, [sidx16], mask=posm))
            match = ((kv & jnp.uint32(0xFFFFFF00)) == prefix) & posm
            dig = lax.convert_element_type(kv & jnp.uint32(0xFF), jnp.int32)
            plsc.addupdate_scatter(hist_v, [lanes * 256 + dig],
                                   ones16, mask=match)
            return 0
        lax.fori_loop(0, trip, histg_body, 0)
        k_rem, d4 = run_scan(k_rem)
        prefix = prefix | d4

        T = prefix
        m0 = jnp.int32(K) - k_rem

        def comp_body(i, carry):
            gt_base, eq_base = carry
            posm = (i * 16 + lanes) < surv_cnt
            sidx16 = surv_v[pl.ds(i * 16, 16)]
            kv = _u(plsc.load_gather(keys_v, [sidx16], mask=posm))
            gt = (kv > T) & posm
            eq = (kv == T) & posm
            ng = plsc.all_reduce_population_count(gt)[0]
            ne = plsc.all_reduce_population_count(eq)[0]

            @pl.when(ng + ne > 0)
            def _():
                kvi = plsc.bitcast(kv, jnp.int32)
                cs_g = plsc.cumsum(jnp.where(gt, 1, 0))
                cs_e = plsc.cumsum(jnp.where(eq, 1, 0))
                gpos = gt_base + cs_g - 1
                epos = m0 + eq_base + cs_e - 1
                plsc.store_scatter(seli_v, [gpos], sidx16, mask=gt)
                plsc.store_scatter(selk_v, [gpos], kvi, mask=gt)
                eqm = eq & (epos < K)
                plsc.store_scatter(seli_v, [epos], sidx16, mask=eqm)
                plsc.store_scatter(selk_v, [epos], kvi, mask=eqm)
            return (gt_base + ng, eq_base + ne)

        lax.fori_loop(0, trip, comp_body, (c_hi, jnp.int32(0)))

        def ext_body(r, _):
            mv = plsc.bitcast(selk_v[pl.ds(0, 16)], jnp.uint32)
            for j in range(1, 13):
                mv = jnp.maximum(
                    mv, plsc.bitcast(selk_v[pl.ds(j * 16, 16)], jnp.uint32))
            M = jnp.max(mv)
            iv = jnp.full((16,), BIGI, jnp.int32)
            for j in range(13):
                kj = plsc.bitcast(selk_v[pl.ds(j * 16, 16)], jnp.uint32)
                ij = seli_v[pl.ds(j * 16, 16)]
                iv = jnp.minimum(iv, jnp.where(kj == M, ij, BIGI))
            I = jnp.min(iv)
            for j in range(13):
                kj = plsc.bitcast(selk_v[pl.ds(j * 16, 16)], jnp.uint32)
                ij = seli_v[pl.ds(j * 16, 16)]
                kill = (kj == M) & (ij == I)
                selk_v[pl.ds(j * 16, 16)] = plsc.bitcast(
                    jnp.where(kill, jnp.uint32(0), kj), jnp.int32)
                seli_v[pl.ds(j * 16, 16)] = jnp.where(kill, BIGI, ij)
            lane0 = lanes == 0
            rvec = jnp.full((16,), r, jnp.int32)
            mvec = plsc.bitcast(jnp.broadcast_to(M, (16,)), jnp.int32)
            ivec = jnp.broadcast_to(jnp.where(I == BIGI, 0, I), (16,))
            plsc.store_scatter(skeys_v, [rvec], mvec, mask=lane0)
            plsc.store_scatter(sidx_v, [rvec], ivec, mask=lane0)
            return 0

        lax.fori_loop(0, K, ext_body, 0, unroll=4)

        for j in range(13):
            u = plsc.bitcast(skeys_v[pl.ds(j * 16, 16)], jnp.uint32)
            top = u >> jnp.uint32(31)
            flip = jnp.where(top == jnp.uint32(1),
                             jnp.uint32(0x80000000), jnp.uint32(0xFFFFFFFF))
            f = plsc.bitcast(u ^ flip, jnp.float32)
            sscore_v[pl.ds(j * 16, 16)] = f
            valid_v[pl.ds(j * 16, 16)] = jnp.where(f > MIN_SCORE, 1, 0)

        pltpu.sync_copy(cls_hbm.at[pl.ds(start, NW)], surv_v)
        mc = jnp.full((16,), -3.0e38, jnp.float32)
        for j in range(13):
            rows = sidx_v[pl.ds(j * 16, 16)]
            cg = plsc.load_gather(surv_v, [rows]) + 1
            clssel_v[pl.ds(j * 16, 16)] = cg
            slot = j * 16 + lanes
            smask = slot < K
            x1 = plsc.load_gather(boxes_v, [rows])
            y1 = plsc.load_gather(boxes_v, [rows + N])
            x2 = plsc.load_gather(boxes_v, [rows + 2 * N])
            y2 = plsc.load_gather(boxes_v, [rows + 3 * N])
            x1_v[pl.ds(j * 16, 16)] = x1
            y1_v[pl.ds(j * 16, 16)] = y1
            x2_v[pl.ds(j * 16, 16)] = x2
            y2_v[pl.ds(j * 16, 16)] = y2
            s4 = slot * 4
            plsc.store_scatter(outb_v, [s4], x1, mask=smask)
            plsc.store_scatter(outb_v, [s4 + 1], y1, mask=smask)
            plsc.store_scatter(outb_v, [s4 + 2], x2, mask=smask)
            plsc.store_scatter(outb_v, [s4 + 3], y2, mask=smask)
            neg = jnp.full((16,), -3.0e38, jnp.float32)
            mc = jnp.maximum(mc, jnp.where(smask, x1, neg))
            mc = jnp.maximum(mc, jnp.where(smask, y1, neg))
            mc = jnp.maximum(mc, jnp.where(smask, x2, neg))
            mc = jnp.maximum(mc, jnp.where(smask, y2, neg))
        off_scale = jnp.max(mc) + jnp.float32(1.0)

        for j in range(13):
            offs = clssel_v[pl.ds(j * 16, 16)].astype(jnp.float32) * off_scale
            x1o = x1_v[pl.ds(j * 16, 16)] + offs
            y1o = y1_v[pl.ds(j * 16, 16)] + offs
            x2o = x2_v[pl.ds(j * 16, 16)] + offs
            y2o = y2_v[pl.ds(j * 16, 16)] + offs
            x1_v[pl.ds(j * 16, 16)] = x1o
            y1_v[pl.ds(j * 16, 16)] = y1o
            x2_v[pl.ds(j * 16, 16)] = x2o
            y2_v[pl.ds(j * 16, 16)] = y2o
            area_v[pl.ds(j * 16, 16)] = (
                jnp.maximum(x2o - x1o, 0.0) * jnp.maximum(y2o - y1o, 0.0))
            sup_v[pl.ds(j * 16, 16)] = zeros16
            keep_v[pl.ds(j * 16, 16)] = zeros16

        def nms_body(r, _):
            kflag = ((sup_v[pl.ds(r, 16)][0] == 0) &
                     (valid_v[pl.ds(r, 16)][0] == 1))
            kvec = jnp.broadcast_to(jnp.where(kflag, 1, 0), (16,))
            plsc.store_scatter(keep_v, [jnp.full((16,), r, jnp.int32)],
                               kvec, mask=lanes == 0)

            @pl.when(kflag)
            def _():
                x1i = x1_v[pl.ds(r, 16)][0]
                y1i = y1_v[pl.ds(r, 16)][0]
                x2i = x2_v[pl.ds(r, 16)][0]
                y2i = y2_v[pl.ds(r, 16)][0]
                ai = area_v[pl.ds(r, 16)][0]

                for j in range(13):
                    c0 = j * 16
                    xx1 = jnp.maximum(x1_v[pl.ds(c0, 16)], x1i)
                    yy1 = jnp.maximum(y1_v[pl.ds(c0, 16)], y1i)
                    xx2 = jnp.minimum(x2_v[pl.ds(c0, 16)], x2i)
                    yy2 = jnp.minimum(y2_v[pl.ds(c0, 16)], y2i)
                    inter = (jnp.maximum(xx2 - xx1, 0.0) *
                             jnp.maximum(yy2 - yy1, 0.0))
                    union = ai + area_v[pl.ds(c0, 16)] - inter
                    iou = inter / jnp.maximum(union, 1e-9)
                    supn = jnp.where(iou > NMS_IOU, 1, 0)
                    sup_v[pl.ds(c0, 16)] = sup_v[pl.ds(c0, 16)] | supn
            return 0

        lax.fori_loop(0, K, nms_body, 0, unroll=4)

        pltpu.sync_copy(outb_v, ob_hbm.at[pl.ds(b * 1024, 1024)])
        pltpu.sync_copy(sscore_v, os_hbm.at[pl.ds(b * OUTW, OUTW)])
        pltpu.sync_copy(clssel_v, oc_hbm.at[pl.ds(b * OUTW, OUTW)])
        pltpu.sync_copy(keep_v, ok_hbm.at[pl.ds(b * OUTW, OUTW)])


def _stage2(keys, cls, boxes):
    mesh = plsc.VectorSubcoreMesh(core_axis_name="c", subcore_axis_name="s",
                                  num_cores=2, num_subcores=16)
    f = functools.partial(
        pl.kernel,
        out_type=[
            jax.ShapeDtypeStruct((B * 1024,), jnp.float32),
            jax.ShapeDtypeStruct((B * OUTW,), jnp.float32),
            jax.ShapeDtypeStruct((B * OUTW,), jnp.int32),
            jax.ShapeDtypeStruct((B * OUTW,), jnp.int32),
        ],
        mesh=mesh,
        compiler_params=pltpu.CompilerParams(needs_layout_passes=False),
        scratch_types=[
            pltpu.VMEM((NW,), jnp.int32),
            pltpu.VMEM((NW,), jnp.int32),
            pltpu.VMEM((N * 4,), jnp.float32),
            pltpu.VMEM((4096,), jnp.int32),
            pltpu.VMEM((SLOTSP,), jnp.int32),
            pltpu.VMEM((SLOTSP,), jnp.int32),
            pltpu.VMEM((SLOTSP,), jnp.int32),
            pltpu.VMEM((SLOTSP,), jnp.int32),
            pltpu.VMEM((OUTW,), jnp.float32),
            pltpu.VMEM((SLOTSP,), jnp.int32),
            pltpu.VMEM((OUTW,), jnp.int32),
            pltpu.VMEM((SLOTSP,), jnp.float32),
            pltpu.VMEM((SLOTSP,), jnp.float32),
            pltpu.VMEM((SLOTSP,), jnp.float32),
            pltpu.VMEM((SLOTSP,), jnp.float32),
            pltpu.VMEM((SLOTSP,), jnp.float32),
            pltpu.VMEM((1024,), jnp.float32),
            pltpu.VMEM((OUTW,), jnp.int32),
            pltpu.VMEM((SLOTSP,), jnp.int32),
        ],
    )(_sc_body)
    return f(keys, cls, boxes)


def kernel(batched_bboxes, batched_scores):
    scores_t = batched_scores.transpose(0, 2, 1)
    keys, cls = _stage1(scores_t)
    boxes_flat = batched_bboxes.transpose(0, 2, 1).reshape(B * N * 4)
    ob, osc, ocl, okp = _stage2(keys, cls, boxes_flat)
    ob = ob.reshape(B, OUTW, 4)[:, :K, :]
    osc = osc.reshape(B, OUTW)[:, :K]
    ocl = ocl.reshape(B, OUTW)[:, :K]
    okp = okp.reshape(B, OUTW)[:, :K]
    return ob, osc, ocl, okp.astype(bool)
